# baseline (speedup 1.0000x reference)
import functools
import numpy as np
import jax
import jax.numpy as jnp
from jax import lax
from jax.experimental import pallas as pl
from jax.experimental.pallas import tpu as pltpu
from jax.experimental.pallas import tpu_sc as plsc

B = 4096
V = 100000
D = 64
DF = 16
NS = 26
NSP = 32
NW = 32
BPW = B // NW
CH = 8
NCH = BPW // CH
SRC = CH * NSP
NSTR = SRC // 128


def _sc_tag_body(sf2, tp, te, sidx, tbuf_a, tbuf_b, tebuf, sem_t):
    wid = lax.axis_index("s") * 2 + lax.axis_index("c")
    base = wid * BPW
    tbuf = [tbuf_a, tbuf_b]
    r0 = pl.multiple_of(wid * (BPW * NSP // 128), 8)
    pltpu.sync_copy(sf2.at[pl.ds(r0, BPW * NSP // 128)], sidx)

    def fetch(c, bi):
        return [pltpu.async_copy(
            tp.at[sidx.at[c * NSTR + s]],
            tbuf[bi].at[pl.ds(s * 128, 128)], sem_t)
            for s in range(NSTR)]

    def pool(c, bi):
        buf = tbuf[bi]

        def pool_row(r, carry):
            for d in range(4):
                acc = buf[r * NSP, pl.ds(d * 16, 16)]
                for j in range(1, NS):
                    acc = acc + buf[r * NSP + j, pl.ds(d * 16, 16)]
                tebuf[r, pl.ds(d * 16, 16)] = acc
            return carry

        lax.fori_loop(0, CH, pool_row, 0)
        pltpu.sync_copy(tebuf, te.at[pl.ds(base + c * CH, CH)])

    pend = fetch(0, 0)
    for c in range(NCH):
        nxt = fetch(c + 1, (c + 1) % 2) if c + 1 < NCH else []
        for t in pend:
            t.wait()
        pool(c, c % 2)
        pend = nxt


_sc_tag = pl.kernel(
    _sc_tag_body,
    out_type=jax.ShapeDtypeStruct((B, 2 * D), jnp.float32),
    mesh=plsc.VectorSubcoreMesh(core_axis_name="c", subcore_axis_name="s"),
    scratch_types=[
        pltpu.VMEM((BPW * NSP // 128, 128), jnp.int32),
        pltpu.VMEM((SRC, 2 * D), jnp.float32),
        pltpu.VMEM((SRC, 2 * D), jnp.float32),
        pltpu.VMEM((CH, 2 * D), jnp.float32),
        pltpu.SemaphoreType.DMA,
    ],
    compiler_params=pltpu.CompilerParams(use_tc_tiling_on_sc=True),
)


def _sc_ui_body(uid2, iid2, cat_ui, ue, ie,
                uidx_v, iidx_v, ubuf, ibuf, sem_u, sem_i):
    wid = lax.axis_index("s") * 2 + lax.axis_index("c")
    base = wid * BPW
    g8 = pl.multiple_of((wid // 8) * 8, 8)
    pltpu.sync_copy(uid2.at[pl.ds(g8, 8)], uidx_v)
    pltpu.sync_copy(iid2.at[pl.ds(g8, 8)], iidx_v)
    row = wid % 8
    cu = pltpu.async_copy(cat_ui.at[uidx_v.at[row]], ubuf, sem_u)
    ci = pltpu.async_copy(cat_ui.at[iidx_v.at[row]], ibuf, sem_i)
    cu.wait()
    ci.wait()
    pltpu.sync_copy(ubuf, ue.at[pl.ds(base, BPW)])
    pltpu.sync_copy(ibuf, ie.at[pl.ds(base, BPW)])


_sc_ui = pl.kernel(
    _sc_ui_body,
    out_type=(jax.ShapeDtypeStruct((B, 2 * D), jnp.float32),) * 2,
    mesh=plsc.VectorSubcoreMesh(core_axis_name="c", subcore_axis_name="s"),
    scratch_types=[
        pltpu.VMEM((8, 128), jnp.int32),
        pltpu.VMEM((8, 128), jnp.int32),
        pltpu.VMEM((BPW, 2 * D), jnp.float32),
        pltpu.VMEM((BPW, 2 * D), jnp.float32),
        pltpu.SemaphoreType.DMA,
        pltpu.SemaphoreType.DMA,
    ],
    compiler_params=pltpu.CompilerParams(use_tc_tiling_on_sc=True),
)

BM = 512


def _mlp_body(u, it, tg, dn, wW, wb, W1, b1, W2, b2, W3, b3, W4, b4, tW, tb,
              out):
    comb = jnp.concatenate([u[...][:, :D], it[...][:, D:], tg[...][:, :D],
                            dn[...]], axis=-1)
    dot = functools.partial(jnp.dot, preferred_element_type=jnp.float32,
                            precision=lax.Precision.HIGHEST)
    wide = dot(comb, wW[...]) + wb[...]
    h = jnp.maximum(dot(comb, W1[...]) + b1[...], 0.0)
    h = jnp.maximum(dot(h, W2[...]) + b2[...], 0.0)
    h = jnp.maximum(dot(h, W3[...]) + b3[...], 0.0)
    deep = dot(h, W4[...]) + b4[...]
    cat2 = jnp.concatenate([wide, deep], axis=-1)
    logit = dot(cat2, tW[...]) + tb[...]
    out[...] = jax.nn.sigmoid(logit)


def _full(shape):
    nd = len(shape)
    return pl.BlockSpec(shape, lambda i: (0,) * nd)


_mlp = pl.pallas_call(
    _mlp_body,
    grid=(B // BM,),
    in_specs=[
        pl.BlockSpec((BM, 2 * D), lambda i: (i, 0)),
        pl.BlockSpec((BM, 2 * D), lambda i: (i, 0)),
        pl.BlockSpec((BM, 2 * D), lambda i: (i, 0)),
        pl.BlockSpec((BM, DF), lambda i: (i, 0)),
        _full((3 * D + DF, D)), _full((D,)),
        _full((3 * D + DF, 2 * D)), _full((2 * D,)),
        _full((2 * D, 2 * D)), _full((2 * D,)),
        _full((2 * D, 2 * D)), _full((2 * D,)),
        _full((2 * D, D)), _full((D,)),
        _full((2 * D, 1)), _full((1,)),
    ],
    out_specs=pl.BlockSpec((BM, 1), lambda i: (i, 0)),
    out_shape=jax.ShapeDtypeStruct((B, 1), jnp.float32),
)


def kernel(user_hashed_ids, item_hashed_ids, dense_features, sparse_features,
           user_tab, item_tab, tag_tab,
           wide_W, wide_b, W1, b1, W2, b2, W3, b3, W4, b4, tog_W, tog_b):
    sf2 = jnp.pad(sparse_features.astype(jnp.int32),
                  ((0, 0), (0, NSP - NS))).reshape(B * NSP // 128, 128)
    uid2 = user_hashed_ids.astype(jnp.int32).reshape(B // 128, 128)
    iid2 = item_hashed_ids.astype(jnp.int32).reshape(B // 128, 128)
    tp = jnp.pad(tag_tab, ((0, 0), (0, D)))
    te = _sc_tag(sf2, tp)
    cat_ui = jnp.concatenate([user_tab, item_tab], axis=1)
    ue, ie = _sc_ui(uid2, iid2, cat_ui)
    return _mlp(ue, ie, te, dense_features,
                wide_W, wide_b, W1, b1, W2, b2, W3, b3, W4, b4, tog_W, tog_b)


# speedup vs baseline: 5.1123x; 5.1123x over previous
import functools
import numpy as np
import jax
import jax.numpy as jnp
from jax import lax
from jax.experimental import pallas as pl
from jax.experimental.pallas import tpu as pltpu
from jax.experimental.pallas import tpu_sc as plsc

B = 4096
V = 100000
D = 64
DF = 16
NS = 26
NW = 32
BPW = B // NW
CH = 16
NCH = BPW // CH
SRC = CH * NS
_STREAMS = [(0, 128), (128, 128), (256, 128), (384, 32)]


def _sc_tag_body(sf_flat, tp, te, tidx_a, tidx_b, tbuf_a, tbuf_b, tebuf,
                 sem_t):
    wid = lax.axis_index("s") * 2 + lax.axis_index("c")
    base = wid * BPW
    tidx = [tidx_a, tidx_b]
    tbuf = [tbuf_a, tbuf_b]

    def fetch(c, bi):
        pltpu.sync_copy(sf_flat.at[pl.ds((base + c * CH) * NS, SRC)],
                        tidx[bi])
        return [pltpu.async_copy(
            tp.at[tidx[bi].at[pl.ds(off, ln)]],
            tbuf[bi].at[pl.ds(off, ln)], sem_t)
            for off, ln in _STREAMS]

    def pool(c, bi):
        buf = tbuf[bi]

        def pool_row(r, carry):
            for d in range(4):
                acc = buf[r * NS, pl.ds(d * 16, 16)]
                for j in range(1, NS):
                    acc = acc + buf[r * NS + j, pl.ds(d * 16, 16)]
                tebuf[r, pl.ds(d * 16, 16)] = acc
            return carry

        lax.fori_loop(0, CH, pool_row, 0)
        pltpu.sync_copy(tebuf, te.at[pl.ds(base + c * CH, CH)])

    pend = fetch(0, 0)
    for c in range(NCH):
        nxt = fetch(c + 1, (c + 1) % 2) if c + 1 < NCH else []
        for t in pend:
            t.wait()
        pool(c, c % 2)
        pend = nxt


_sc_tag = pl.kernel(
    _sc_tag_body,
    out_type=jax.ShapeDtypeStruct((B, 2 * D), jnp.float32),
    mesh=plsc.VectorSubcoreMesh(core_axis_name="c", subcore_axis_name="s"),
    scratch_types=[
        pltpu.VMEM((SRC,), jnp.int32),
        pltpu.VMEM((SRC,), jnp.int32),
        pltpu.VMEM((SRC, 2 * D), jnp.float32),
        pltpu.VMEM((SRC, 2 * D), jnp.float32),
        pltpu.VMEM((CH, 2 * D), jnp.float32),
        pltpu.SemaphoreType.DMA,
    ],
    compiler_params=pltpu.CompilerParams(use_tc_tiling_on_sc=True),
)


def _sc_ui_body(uid2, iid2, cat_ui, ue, ie,
                uidx_v, iidx_v, ubuf, ibuf, sem_u, sem_i):
    wid = lax.axis_index("s") * 2 + lax.axis_index("c")
    base = wid * BPW
    g8 = pl.multiple_of((wid // 8) * 8, 8)
    pltpu.sync_copy(uid2.at[pl.ds(g8, 8)], uidx_v)
    pltpu.sync_copy(iid2.at[pl.ds(g8, 8)], iidx_v)
    row = wid % 8
    cu = pltpu.async_copy(cat_ui.at[uidx_v.at[row]], ubuf, sem_u)
    ci = pltpu.async_copy(cat_ui.at[iidx_v.at[row]], ibuf, sem_i)
    cu.wait()
    ci.wait()
    pltpu.sync_copy(ubuf, ue.at[pl.ds(base, BPW)])
    pltpu.sync_copy(ibuf, ie.at[pl.ds(base, BPW)])


_sc_ui = pl.kernel(
    _sc_ui_body,
    out_type=(jax.ShapeDtypeStruct((B, 2 * D), jnp.float32),) * 2,
    mesh=plsc.VectorSubcoreMesh(core_axis_name="c", subcore_axis_name="s"),
    scratch_types=[
        pltpu.VMEM((8, 128), jnp.int32),
        pltpu.VMEM((8, 128), jnp.int32),
        pltpu.VMEM((BPW, 2 * D), jnp.float32),
        pltpu.VMEM((BPW, 2 * D), jnp.float32),
        pltpu.SemaphoreType.DMA,
        pltpu.SemaphoreType.DMA,
    ],
    compiler_params=pltpu.CompilerParams(use_tc_tiling_on_sc=True),
)

BM = 512


def _mlp_body(u, it, tg, dn, wW, wb, W1, b1, W2, b2, W3, b3, W4, b4, tW, tb,
              out):
    comb = jnp.concatenate([u[...][:, :D], it[...][:, D:], tg[...][:, :D],
                            dn[...]], axis=-1)
    dot = functools.partial(jnp.dot, preferred_element_type=jnp.float32,
                            precision=lax.Precision.HIGHEST)
    wide = dot(comb, wW[...]) + wb[...]
    h = jnp.maximum(dot(comb, W1[...]) + b1[...], 0.0)
    h = jnp.maximum(dot(h, W2[...]) + b2[...], 0.0)
    h = jnp.maximum(dot(h, W3[...]) + b3[...], 0.0)
    deep = dot(h, W4[...]) + b4[...]
    cat2 = jnp.concatenate([wide, deep], axis=-1)
    logit = dot(cat2, tW[...]) + tb[...]
    out[...] = jax.nn.sigmoid(logit)


def _full(shape):
    nd = len(shape)
    return pl.BlockSpec(shape, lambda i: (0,) * nd)


_mlp = pl.pallas_call(
    _mlp_body,
    grid=(B // BM,),
    in_specs=[
        pl.BlockSpec((BM, 2 * D), lambda i: (i, 0)),
        pl.BlockSpec((BM, 2 * D), lambda i: (i, 0)),
        pl.BlockSpec((BM, 2 * D), lambda i: (i, 0)),
        pl.BlockSpec((BM, DF), lambda i: (i, 0)),
        _full((3 * D + DF, D)), _full((D,)),
        _full((3 * D + DF, 2 * D)), _full((2 * D,)),
        _full((2 * D, 2 * D)), _full((2 * D,)),
        _full((2 * D, 2 * D)), _full((2 * D,)),
        _full((2 * D, D)), _full((D,)),
        _full((2 * D, 1)), _full((1,)),
    ],
    out_specs=pl.BlockSpec((BM, 1), lambda i: (i, 0)),
    out_shape=jax.ShapeDtypeStruct((B, 1), jnp.float32),
)


def kernel(user_hashed_ids, item_hashed_ids, dense_features, sparse_features,
           user_tab, item_tab, tag_tab,
           wide_W, wide_b, W1, b1, W2, b2, W3, b3, W4, b4, tog_W, tog_b):
    sf_flat = sparse_features.astype(jnp.int32).reshape(-1)
    uid2 = user_hashed_ids.astype(jnp.int32).reshape(B // 128, 128)
    iid2 = item_hashed_ids.astype(jnp.int32).reshape(B // 128, 128)
    tp = jnp.pad(tag_tab, ((0, 0), (0, D)))
    te = _sc_tag(sf_flat, tp)
    cat_ui = jnp.concatenate([user_tab, item_tab], axis=1)
    ue, ie = _sc_ui(uid2, iid2, cat_ui)
    return _mlp(ue, ie, te, dense_features,
                wide_W, wide_b, W1, b1, W2, b2, W3, b3, W4, b4, tog_W, tog_b)


# speedup vs baseline: 5.2544x; 1.0278x over previous
import functools
import numpy as np
import jax
import jax.numpy as jnp
from jax import lax
from jax.experimental import pallas as pl
from jax.experimental.pallas import tpu as pltpu
from jax.experimental.pallas import tpu_sc as plsc

B = 4096
V = 100000
D = 64
DF = 16
NS = 26
NW = 32
BPW = B // NW
CH = 16
NCH = BPW // CH
SRC = CH * NS
_STREAMS = [(0, 128), (128, 128), (256, 128), (384, 32)]


def _sc_tag_body(sf_flat, tp, te, tidx_a, tidx_b, tbuf_a, tbuf_b, tebuf,
                 sem_t):
    wid = lax.axis_index("s") * 2 + lax.axis_index("c")
    base = wid * BPW
    tidx = [tidx_a, tidx_b]
    tbuf = [tbuf_a, tbuf_b]

    def fetch(c, bi):
        pltpu.sync_copy(sf_flat.at[pl.ds((base + c * CH) * NS, SRC)],
                        tidx[bi])
        return [pltpu.async_copy(
            tp.at[tidx[bi].at[pl.ds(off, ln)]],
            tbuf[bi].at[pl.ds(off, ln)], sem_t)
            for off, ln in _STREAMS]

    def pool(c, bi):
        buf = tbuf[bi]

        def pool_row(r, carry):
            for d in range(4):
                acc = buf[r * NS, pl.ds(d * 16, 16)]
                for j in range(1, NS):
                    acc = acc + buf[r * NS + j, pl.ds(d * 16, 16)]
                tebuf[r, pl.ds(d * 16, 16)] = acc
            return carry

        lax.fori_loop(0, CH, pool_row, 0)
        pltpu.sync_copy(tebuf, te.at[pl.ds(base + c * CH, CH)])

    pend = fetch(0, 0)
    for c in range(NCH):
        nxt = fetch(c + 1, (c + 1) % 2) if c + 1 < NCH else []
        for t in pend:
            t.wait()
        pool(c, c % 2)
        pend = nxt


_sc_tag = pl.kernel(
    _sc_tag_body,
    out_type=jax.ShapeDtypeStruct((B, 2 * D), jnp.float32),
    mesh=plsc.VectorSubcoreMesh(core_axis_name="c", subcore_axis_name="s"),
    scratch_types=[
        pltpu.VMEM((SRC,), jnp.int32),
        pltpu.VMEM((SRC,), jnp.int32),
        pltpu.VMEM((SRC, 2 * D), jnp.float32),
        pltpu.VMEM((SRC, 2 * D), jnp.float32),
        pltpu.VMEM((CH, 2 * D), jnp.float32),
        pltpu.SemaphoreType.DMA,
    ],
    compiler_params=pltpu.CompilerParams(use_tc_tiling_on_sc=True),
)


def _sc_ui_body(uid2, iid2, cat_ui, ue, ie,
                uidx_v, iidx_v, ubuf, ibuf, sem_u, sem_i):
    wid = lax.axis_index("s") * 2 + lax.axis_index("c")
    base = wid * BPW
    g8 = pl.multiple_of((wid // 8) * 8, 8)
    pltpu.sync_copy(uid2.at[pl.ds(g8, 8)], uidx_v)
    pltpu.sync_copy(iid2.at[pl.ds(g8, 8)], iidx_v)
    row = wid % 8
    cu = pltpu.async_copy(cat_ui.at[uidx_v.at[row]], ubuf, sem_u)
    ci = pltpu.async_copy(cat_ui.at[iidx_v.at[row]], ibuf, sem_i)
    cu.wait()
    ci.wait()
    pltpu.sync_copy(ubuf, ue.at[pl.ds(base, BPW)])
    pltpu.sync_copy(ibuf, ie.at[pl.ds(base, BPW)])


_sc_ui = pl.kernel(
    _sc_ui_body,
    out_type=(jax.ShapeDtypeStruct((B, 2 * D), jnp.float32),) * 2,
    mesh=plsc.VectorSubcoreMesh(core_axis_name="c", subcore_axis_name="s"),
    scratch_types=[
        pltpu.VMEM((8, 128), jnp.int32),
        pltpu.VMEM((8, 128), jnp.int32),
        pltpu.VMEM((BPW, 2 * D), jnp.float32),
        pltpu.VMEM((BPW, 2 * D), jnp.float32),
        pltpu.SemaphoreType.DMA,
        pltpu.SemaphoreType.DMA,
    ],
    compiler_params=pltpu.CompilerParams(use_tc_tiling_on_sc=True),
)

BM = 1024


def _mlp_body(u, it, tg, dn, wv, W1, b1, W2, b2, W3, b3, W4v, bc, out):
    comb = jnp.concatenate([u[...][:, :D], it[...][:, D:], tg[...][:, :D],
                            dn[...]], axis=-1)
    dot = functools.partial(jnp.dot, preferred_element_type=jnp.float32,
                            precision=lax.Precision.HIGHEST)
    h = jnp.maximum(dot(comb, W1[...]) + b1[...], 0.0)
    h = jnp.maximum(dot(h, W2[...]) + b2[...], 0.0)
    h = jnp.maximum(dot(h, W3[...]) + b3[...], 0.0)
    logit = dot(comb, wv[...]) + dot(h, W4v[...]) + bc[...]
    out[...] = jax.nn.sigmoid(logit)


def _full(shape):
    nd = len(shape)
    return pl.BlockSpec(shape, lambda i: (0,) * nd)


_mlp = pl.pallas_call(
    _mlp_body,
    grid=(B // BM,),
    in_specs=[
        pl.BlockSpec((BM, 2 * D), lambda i: (i, 0)),
        pl.BlockSpec((BM, 2 * D), lambda i: (i, 0)),
        pl.BlockSpec((BM, 2 * D), lambda i: (i, 0)),
        pl.BlockSpec((BM, DF), lambda i: (i, 0)),
        _full((3 * D + DF, 1)),
        _full((3 * D + DF, 2 * D)), _full((2 * D,)),
        _full((2 * D, 2 * D)), _full((2 * D,)),
        _full((2 * D, 2 * D)), _full((2 * D,)),
        _full((2 * D, 1)), _full((1,)),
    ],
    out_specs=pl.BlockSpec((BM, 1), lambda i: (i, 0)),
    out_shape=jax.ShapeDtypeStruct((B, 1), jnp.float32),
)


def kernel(user_hashed_ids, item_hashed_ids, dense_features, sparse_features,
           user_tab, item_tab, tag_tab,
           wide_W, wide_b, W1, b1, W2, b2, W3, b3, W4, b4, tog_W, tog_b):
    sf_flat = sparse_features.astype(jnp.int32).reshape(-1)
    uid2 = user_hashed_ids.astype(jnp.int32).reshape(B // 128, 128)
    iid2 = item_hashed_ids.astype(jnp.int32).reshape(B // 128, 128)
    tp = jnp.pad(tag_tab, ((0, 0), (0, D)))
    te = _sc_tag(sf_flat, tp)
    cat_ui = jnp.concatenate([user_tab, item_tab], axis=1)
    ue, ie = _sc_ui(uid2, iid2, cat_ui)
    wv = wide_W @ tog_W[:D]
    W4v = W4 @ tog_W[D:]
    bc = wide_b @ tog_W[:D] + b4 @ tog_W[D:] + tog_b
    return _mlp(ue, ie, te, dense_features,
                wv, W1, b1, W2, b2, W3, b3, W4v, bc)


# speedup vs baseline: 5.6107x; 1.0678x over previous
import functools
import numpy as np
import jax
import jax.numpy as jnp
from jax import lax
from jax.experimental import pallas as pl
from jax.experimental.pallas import tpu as pltpu
from jax.experimental.pallas import tpu_sc as plsc

B = 4096
V = 100000
D = 64
DF = 16
NS = 26
NW = 32
BPW = B // NW
CH = 16
NCH = BPW // CH
SRC = CH * NS
_STREAMS = [(0, 128), (128, 128), (256, 128), (384, 32)]


def _sc_tag_body(sf_flat, tp, te, tidx_a, tidx_b, tbuf_a, tbuf_b, tebuf,
                 sem_t):
    wid = lax.axis_index("s") * 2 + lax.axis_index("c")
    base = wid * BPW
    tidx = [tidx_a, tidx_b]
    tbuf = [tbuf_a, tbuf_b]

    def fetch(c, bi):
        pltpu.sync_copy(sf_flat.at[pl.ds((base + c * CH) * NS, SRC)],
                        tidx[bi])
        return [pltpu.async_copy(
            tp.at[tidx[bi].at[pl.ds(off, ln)]],
            tbuf[bi].at[pl.ds(off, ln)], sem_t)
            for off, ln in _STREAMS]

    def pool(c, bi):
        buf = tbuf[bi]

        def pool_row(r, carry):
            for d in range(4):
                acc = buf[r * NS, pl.ds(d * 16, 16)]
                for j in range(1, NS):
                    acc = acc + buf[r * NS + j, pl.ds(d * 16, 16)]
                tebuf[r, pl.ds(d * 16, 16)] = acc
            return carry

        lax.fori_loop(0, CH, pool_row, 0)
        pltpu.sync_copy(tebuf, te.at[pl.ds(base + c * CH, CH)])

    pend = fetch(0, 0)
    for c in range(NCH):
        nxt = fetch(c + 1, (c + 1) % 2) if c + 1 < NCH else []
        for t in pend:
            t.wait()
        pool(c, c % 2)
        pend = nxt


_sc_tag = pl.kernel(
    _sc_tag_body,
    out_type=jax.ShapeDtypeStruct((B, 2 * D), jnp.float32),
    mesh=plsc.VectorSubcoreMesh(core_axis_name="c", subcore_axis_name="s"),
    scratch_types=[
        pltpu.VMEM((SRC,), jnp.int32),
        pltpu.VMEM((SRC,), jnp.int32),
        pltpu.VMEM((SRC, 2 * D), jnp.float32),
        pltpu.VMEM((SRC, 2 * D), jnp.float32),
        pltpu.VMEM((CH, 2 * D), jnp.float32),
        pltpu.SemaphoreType.DMA,
    ],
    compiler_params=pltpu.CompilerParams(use_tc_tiling_on_sc=True),
)


def _sc_ui_body(uid2, iid2, cat_ui, ue, ie,
                uidx_v, iidx_v, ubuf, ibuf, sem_u, sem_i):
    wid = lax.axis_index("s") * 2 + lax.axis_index("c")
    base = wid * BPW
    g8 = pl.multiple_of((wid // 8) * 8, 8)
    pltpu.sync_copy(uid2.at[pl.ds(g8, 8)], uidx_v)
    pltpu.sync_copy(iid2.at[pl.ds(g8, 8)], iidx_v)
    row = wid % 8
    cu = pltpu.async_copy(cat_ui.at[uidx_v.at[row]], ubuf, sem_u)
    ci = pltpu.async_copy(cat_ui.at[iidx_v.at[row]], ibuf, sem_i)
    cu.wait()
    ci.wait()
    pltpu.sync_copy(ubuf, ue.at[pl.ds(base, BPW)])
    pltpu.sync_copy(ibuf, ie.at[pl.ds(base, BPW)])


_sc_ui = pl.kernel(
    _sc_ui_body,
    out_type=(jax.ShapeDtypeStruct((B, 2 * D), jnp.float32),) * 2,
    mesh=plsc.VectorSubcoreMesh(core_axis_name="c", subcore_axis_name="s"),
    scratch_types=[
        pltpu.VMEM((8, 128), jnp.int32),
        pltpu.VMEM((8, 128), jnp.int32),
        pltpu.VMEM((BPW, 2 * D), jnp.float32),
        pltpu.VMEM((BPW, 2 * D), jnp.float32),
        pltpu.SemaphoreType.DMA,
        pltpu.SemaphoreType.DMA,
    ],
    compiler_params=pltpu.CompilerParams(use_tc_tiling_on_sc=True),
)

BM = 1024


def _mlp_body(u, it, tg, dn, wv, W1, b1, W2, b2, W3, b3, W4v, bc, out):
    comb = jnp.concatenate([u[...][:, :D], it[...][:, D:], tg[...][:, :D],
                            dn[...]], axis=-1)
    dot = functools.partial(jnp.dot, preferred_element_type=jnp.float32,
                            precision=lax.Precision.DEFAULT)
    h = jnp.maximum(dot(comb, W1[...]) + b1[...], 0.0)
    h = jnp.maximum(dot(h, W2[...]) + b2[...], 0.0)
    h = jnp.maximum(dot(h, W3[...]) + b3[...], 0.0)
    logit = dot(comb, wv[...]) + dot(h, W4v[...]) + bc[...]
    out[...] = jax.nn.sigmoid(logit)


def _full(shape):
    nd = len(shape)
    return pl.BlockSpec(shape, lambda i: (0,) * nd)


_mlp = pl.pallas_call(
    _mlp_body,
    grid=(B // BM,),
    in_specs=[
        pl.BlockSpec((BM, 2 * D), lambda i: (i, 0)),
        pl.BlockSpec((BM, 2 * D), lambda i: (i, 0)),
        pl.BlockSpec((BM, 2 * D), lambda i: (i, 0)),
        pl.BlockSpec((BM, DF), lambda i: (i, 0)),
        _full((3 * D + DF, 1)),
        _full((3 * D + DF, 2 * D)), _full((2 * D,)),
        _full((2 * D, 2 * D)), _full((2 * D,)),
        _full((2 * D, 2 * D)), _full((2 * D,)),
        _full((2 * D, 1)), _full((1,)),
    ],
    out_specs=pl.BlockSpec((BM, 1), lambda i: (i, 0)),
    out_shape=jax.ShapeDtypeStruct((B, 1), jnp.float32),
)


def kernel(user_hashed_ids, item_hashed_ids, dense_features, sparse_features,
           user_tab, item_tab, tag_tab,
           wide_W, wide_b, W1, b1, W2, b2, W3, b3, W4, b4, tog_W, tog_b):
    sf_flat = sparse_features.astype(jnp.int32).reshape(-1)
    uid2 = user_hashed_ids.astype(jnp.int32).reshape(B // 128, 128)
    iid2 = item_hashed_ids.astype(jnp.int32).reshape(B // 128, 128)
    tp = jnp.pad(tag_tab, ((0, 0), (0, D)))
    te = _sc_tag(sf_flat, tp)
    cat_ui = jnp.concatenate([user_tab, item_tab], axis=1)
    ue, ie = _sc_ui(uid2, iid2, cat_ui)
    wv = wide_W @ tog_W[:D]
    W4v = W4 @ tog_W[D:]
    bc = wide_b @ tog_W[:D] + b4 @ tog_W[D:] + tog_b
    return _mlp(ue, ie, te, dense_features,
                wv, W1, b1, W2, b2, W3, b3, W4v, bc)
